# baseline (speedup 1.0000x reference)
import functools

import jax
import jax.numpy as jnp
from jax import lax
from jax.experimental import pallas as pl
from jax.experimental.pallas import tpu as pltpu
from jax.experimental.pallas import tpu_sc as plsc

B, T, NC, DC = 1024, 50, 26, 16
V, D = 1000000, 64
NW = 32
B_W = B // NW
CH = T // 2
ROWS = CH * NC
DOUT = D + DC


def _emb_body(table, cat3, cont3, out3, ib0, ib1, rows0, rows1, out0, out1,
              g0, g1, w0, w1, i0, i1):
    wid = lax.axis_index("s") * 2 + lax.axis_index("c")
    b_base = wid * B_W

    ibuf = (ib0, ib1)
    rows_b = (rows0, rows1)
    out_b = (out0, out1)
    gsem = (g0, g1)
    wsem = (w0, w1)
    isem = (i0, i1)

    def fire_gathers(h, bp, par):
        for t in range(CH):
            pltpu.async_copy(
                table.at[ibuf[bp].at[h * CH + t]],
                rows_b[par].at[pl.ds(t * NC, NC)], gsem[par])

    def drain_gathers(par):
        pltpu.make_async_copy(
            table.at[pl.ds(0, ROWS)], rows_b[par], gsem[par]).wait()

    pltpu.sync_copy(cat3.at[b_base], ib0)
    fire_gathers(0, 0, 0)
    fire_gathers(1, 0, 1)

    @pl.loop(0, B_W // 2)
    def _(g16):
        for bp in range(2):
            b = g16 * 2 + bp
            for h in range(2):
                par = h
                rows_v = rows_b[par]
                out_v = out_b[par]

                @pl.when(b >= 1)
                def _():
                    pltpu.make_async_copy(
                        out_v, out3.at[b_base + b, pl.ds(h * CH, CH)],
                        wsem[par]).wait()

                pltpu.sync_copy(cont3.at[b_base + b, pl.ds(h * CH, CH)],
                                out_v.at[:, pl.ds(D, DC)])

                if h == 0:
                    @pl.when(b + 1 < B_W)
                    def _():
                        pltpu.async_copy(cat3.at[b_base + b + 1],
                                         ibuf[1 - bp], isem[1 - bp])

                drain_gathers(par)

                @plsc.parallel_loop(0, CH)
                def _(p):
                    r0 = p * NC
                    for v in range(D // 16):
                        sl = pl.ds(v * 16, 16)
                        acc = rows_v[r0, sl]
                        for cc in range(1, NC):
                            acc = acc + rows_v[r0 + cc, sl]
                        out_v[p, sl] = acc

                if h == 0:
                    @pl.when(b + 1 < B_W)
                    def _():
                        pltpu.make_async_copy(cat3.at[b_base], ibuf[1 - bp],
                                              isem[1 - bp]).wait()

                @pl.when(b + 1 < B_W)
                def _():
                    fire_gathers(h, 1 - bp, par)

                pltpu.async_copy(
                    out_v, out3.at[b_base + b, pl.ds(h * CH, CH)], wsem[par])

    pltpu.make_async_copy(
        out0, out3.at[b_base, pl.ds(0, CH)], w0).wait()
    pltpu.make_async_copy(
        out1, out3.at[b_base, pl.ds(CH, CH)], w1).wait()


@jax.jit
def _embed_concat(table, cat3, cont3):
    mesh = plsc.VectorSubcoreMesh(core_axis_name="c", subcore_axis_name="s")
    kern = functools.partial(
        pl.kernel,
        mesh=mesh,
        out_type=jax.ShapeDtypeStruct((B, T, DOUT), jnp.float32),
        scratch_types=[
            pltpu.VMEM((T, NC), jnp.int32),
            pltpu.VMEM((T, NC), jnp.int32),
            pltpu.VMEM((ROWS, D), jnp.float32),
            pltpu.VMEM((ROWS, D), jnp.float32),
            pltpu.VMEM((CH, DOUT), jnp.float32),
            pltpu.VMEM((CH, DOUT), jnp.float32),
            pltpu.SemaphoreType.DMA,
            pltpu.SemaphoreType.DMA,
            pltpu.SemaphoreType.DMA,
            pltpu.SemaphoreType.DMA,
            pltpu.SemaphoreType.DMA,
            pltpu.SemaphoreType.DMA,
        ],
        compiler_params=pltpu.CompilerParams(use_tc_tiling_on_sc=False),
    )(_emb_body)
    return kern(table, cat3, cont3)


def kernel(ContTensor, CatTensor, LabelTensor, DoseTensor, TimeDiffTensor,
           VTensor, VancoElTensor, PtList, LengList, embed_weight):
    outEmb = _embed_concat(embed_weight, CatTensor.astype(jnp.int32),
                           ContTensor)
    return (outEmb, LabelTensor, LengList, DoseTensor, TimeDiffTensor,
            VTensor, VancoElTensor, PtList)


# speedup vs baseline: 1.0953x; 1.0953x over previous
import functools

import jax
import jax.numpy as jnp
from jax import lax
from jax.experimental import pallas as pl
from jax.experimental.pallas import tpu as pltpu
from jax.experimental.pallas import tpu_sc as plsc

B, T, NC, DC = 1024, 50, 26, 16
V, D = 1000000, 64
P = B * T
NW = 32
B_W = B // NW
P_W = P // NW
IDX_W = P_W * NC
CH = 16
N_CH = P_W // CH
ROWS = CH * NC


def _flatten_body(cat3, catf, v3, vf):
    wid = lax.axis_index("s") * 2 + lax.axis_index("c")
    b_base = wid * B_W

    @pl.loop(0, B_W // 2)
    def _(j):
        b = b_base + 2 * j
        pltpu.sync_copy(cat3.at[pl.ds(b, 2)], v3)
        for bb in range(2):
            @plsc.parallel_loop(0, T)
            def _(t):
                off = (bb * T + t) * NC
                vf[pl.ds(off, 16)] = v3[bb, t, pl.ds(0, 16)]
                vf[pl.ds(off + NC - 16, 16)] = v3[bb, t, pl.ds(NC - 16, 16)]
        pltpu.sync_copy(vf, catf.at[pl.ds(b * T * NC, 2 * T * NC)])


def _emb_body(table, idx, out, idx_v, rows0, rows1, out0, out1,
              g0, g1, w0, w1):
    wid = lax.axis_index("s") * 2 + lax.axis_index("c")
    pos_base = wid * P_W
    pltpu.sync_copy(idx.at[pl.ds(pos_base * NC, IDX_W)], idx_v)

    rows_b = (rows0, rows1)
    out_b = (out0, out1)
    gsem = (g0, g1)
    wsem = (w0, w1)

    def start_gather(c, par):
        pltpu.async_copy(
            table.at[idx_v.at[pl.ds(c * ROWS, ROWS)]], rows_b[par], gsem[par])

    start_gather(0, 0)
    start_gather(1, 1)

    @pl.loop(0, N_CH // 2)
    def _(g2):
        for par in range(2):
            c = g2 * 2 + par
            pos0 = pos_base + c * CH
            rows_v = rows_b[par]
            out_v = out_b[par]

            @pl.when(c >= 2)
            def _():
                pltpu.make_async_copy(
                    out_v, out.at[pl.ds(pos0, CH)], wsem[par]).wait()

            pltpu.make_async_copy(
                table.at[idx_v.at[pl.ds(c * ROWS, ROWS)]], rows_v,
                gsem[par]).wait()

            @plsc.parallel_loop(0, CH)
            def _(p):
                r0 = p * NC
                for v in range(D // 16):
                    sl = pl.ds(v * 16, 16)
                    acc = rows_v[r0, sl]
                    for cc in range(1, NC):
                        acc = acc + rows_v[r0 + cc, sl]
                    out_v[p, sl] = acc

            @pl.when(c + 2 < N_CH)
            def _():
                start_gather(c + 2, par)

            pltpu.async_copy(out_v, out.at[pl.ds(pos0, CH)], wsem[par])

    pltpu.make_async_copy(out0, out.at[pl.ds(pos_base, CH)], w0).wait()
    pltpu.make_async_copy(out1, out.at[pl.ds(pos_base, CH)], w1).wait()


@jax.jit
def _embed_sum(table, cat3):
    mesh = plsc.VectorSubcoreMesh(core_axis_name="c", subcore_axis_name="s")
    flatten = functools.partial(
        pl.kernel,
        mesh=mesh,
        out_type=jax.ShapeDtypeStruct((P * NC,), jnp.int32),
        scratch_types=[
            pltpu.VMEM((2, T, NC), jnp.int32),
            pltpu.VMEM((2 * T * NC,), jnp.int32),
        ],
        compiler_params=pltpu.CompilerParams(use_tc_tiling_on_sc=True),
    )(_flatten_body)
    catf = flatten(cat3)

    kern = functools.partial(
        pl.kernel,
        mesh=mesh,
        out_type=jax.ShapeDtypeStruct((P, D), jnp.float32),
        scratch_types=[
            pltpu.VMEM((IDX_W,), jnp.int32),
            pltpu.VMEM((ROWS, D), jnp.float32),
            pltpu.VMEM((ROWS, D), jnp.float32),
            pltpu.VMEM((CH, D), jnp.float32),
            pltpu.VMEM((CH, D), jnp.float32),
            pltpu.SemaphoreType.DMA,
            pltpu.SemaphoreType.DMA,
            pltpu.SemaphoreType.DMA,
            pltpu.SemaphoreType.DMA,
        ],
        compiler_params=pltpu.CompilerParams(use_tc_tiling_on_sc=False),
    )(_emb_body)
    return kern(table, catf)


def kernel(ContTensor, CatTensor, LabelTensor, DoseTensor, TimeDiffTensor,
           VTensor, VancoElTensor, PtList, LengList, embed_weight):
    sum2 = _embed_sum(embed_weight, CatTensor.astype(jnp.int32))
    outEmb = jnp.concatenate([sum2.reshape(B, T, D), ContTensor], axis=2)
    return (outEmb, LabelTensor, LengList, DoseTensor, TimeDiffTensor,
            VTensor, VancoElTensor, PtList)
